# baseline (speedup 1.0000x reference)
import functools

import jax
import jax.numpy as jnp
from jax import lax
from jax.experimental import pallas as pl
from jax.experimental.pallas import tpu as pltpu
from jax.experimental.pallas import tpu_sc as plsc

_NC = 2
_NS = 16
_EB = 128
_DEGW = 16
_RB = 5120


def _sc_deg(np_total, num_chunks):
  rows_per_tile = np_total // _NS
  mesh = plsc.VectorSubcoreMesh(core_axis_name="c", subcore_axis_name="s")
  ntg = _NC * _NS
  maxc_s = num_chunks - (num_chunks * (ntg - 1)) // ntg

  scratch = [
      pltpu.VMEM((maxc_s, _EB), jnp.int32),
      pltpu.VMEM((_EB, _DEGW), jnp.float32),
      pltpu.VMEM((rows_per_tile, _DEGW), jnp.float32),
      pltpu.VMEM_SHARED((np_total, _DEGW), jnp.float32),
      pltpu.SemaphoreType.DMA,
      pltpu.SemaphoreType.DMA,
  ]

  nt = _NC * _NS
  maxc = num_chunks - (num_chunks * (nt - 1)) // nt

  def body(ei3, zrows, orows, out, dst_all, rows_v, stage_v, acc_sh,
           ssa, ssb):
    cid = lax.axis_index("c")
    sid = lax.axis_index("s")
    tid = cid * _NS + sid
    row0 = sid * rows_per_tile
    tile_rows = pl.ds(row0, rows_per_tile)
    start = (num_chunks * tid) // nt
    cnt = (num_chunks * (tid + 1)) // nt - start

    pltpu.sync_copy(zrows, stage_v)
    pltpu.sync_copy(stage_v, acc_sh.at[tile_rows])
    pltpu.sync_copy(orows, rows_v)
    pltpu.sync_copy(ei3.at[1, pl.ds(start, maxc)], dst_all)
    plsc.subcore_barrier()

    def scat(c, sem):
      pltpu.async_copy(rows_v, acc_sh.at[dst_all.at[c]], sem, add=True)

    def scat_wait(sem):
      pltpu.make_async_copy(rows_v, acc_sh.at[dst_all.at[0]], sem).wait()

    def pair(j, carry):
      c0 = 2 * j

      @pl.when(j > 0)
      def _():
        scat_wait(ssa)
      scat(c0, ssa)

      @pl.when(j > 0)
      def _():
        scat_wait(ssb)
      scat(c0 + 1, ssb)
      return carry

    lax.fori_loop(0, cnt // 2, pair, 0)

    @pl.when(cnt % 2 == 1)
    def _():
      scat_wait(ssa)
      scat(cnt - 1, ssa)
    scat_wait(ssa)
    scat_wait(ssb)
    plsc.subcore_barrier()

    pltpu.sync_copy(acc_sh.at[tile_rows], stage_v)
    pltpu.sync_copy(stage_v, out.at[cid, tile_rows])

  return pl.kernel(
      body,
      out_type=jax.ShapeDtypeStruct((_NC, np_total, _DEGW), jnp.float32),
      mesh=mesh,
      scratch_types=scratch,
      compiler_params=pltpu.CompilerParams(use_tc_tiling_on_sc=False),
  )


def _sc_agg(np_total, wh, num_chunks):
  mesh = plsc.VectorSubcoreMesh(core_axis_name="c", subcore_axis_name="s")

  maxc = num_chunks - (num_chunks * (_NS - 1)) // _NS

  scratch = [
      pltpu.VMEM((maxc, _EB), jnp.int32),
      pltpu.VMEM((maxc, _EB), jnp.int32),
      pltpu.VMEM((_EB, wh), jnp.float32),
      pltpu.VMEM((_EB, wh), jnp.float32),
      pltpu.VMEM_SHARED((np_total, wh), jnp.float32),
      pltpu.VMEM_SHARED((np_total, wh), jnp.float32),
      pltpu.SemaphoreType.DMA,
      pltpu.SemaphoreType.DMA,
      pltpu.SemaphoreType.DMA,
      pltpu.SemaphoreType.DMA,
  ]

  def body(table, ei3, zacc, out,
           src_all, dst_all, rows_a, rows_b, table_sh, acc_sh,
           gsa, gsb, ssa, ssb):
    cid = lax.axis_index("c")
    sid = lax.axis_index("s")
    cols = pl.ds(cid * wh, wh)
    start = (num_chunks * sid) // _NS
    cnt = (num_chunks * (sid + 1)) // _NS - start

    @pl.when(sid == 0)
    def _():
      pltpu.sync_copy(zacc, acc_sh)
      pltpu.sync_copy(table.at[:, cols], table_sh)

    pltpu.sync_copy(ei3.at[0, pl.ds(start, maxc)], src_all)
    pltpu.sync_copy(ei3.at[1, pl.ds(start, maxc)], dst_all)
    plsc.subcore_barrier()

    def gath(c, rows, sem):
      pltpu.async_copy(table_sh.at[src_all.at[c]], rows, sem)

    def gath_wait(rows, sem):
      pltpu.make_async_copy(table_sh.at[src_all.at[0]], rows, sem).wait()

    def scat(c, rows, sem):
      pltpu.async_copy(rows, acc_sh.at[dst_all.at[c]], sem, add=True)

    def scat_wait(rows, sem):
      pltpu.make_async_copy(rows, acc_sh.at[dst_all.at[0]], sem).wait()

    def pair(j, carry):
      c0 = 2 * j
      c1 = c0 + 1

      @pl.when(j > 0)
      def _():
        scat_wait(rows_a, ssa)
      gath(c0, rows_a, gsa)

      @pl.when(j > 0)
      def _():
        scat_wait(rows_b, ssb)
      gath(c1, rows_b, gsb)

      gath_wait(rows_a, gsa)
      scat(c0, rows_a, ssa)
      gath_wait(rows_b, gsb)
      scat(c1, rows_b, ssb)
      return carry

    lax.fori_loop(0, cnt // 2, pair, 0)

    @pl.when(cnt % 2 == 1)
    def _():
      scat_wait(rows_a, ssa)
      gath(cnt - 1, rows_a, gsa)
      gath_wait(rows_a, gsa)
      scat(cnt - 1, rows_a, ssa)
    scat_wait(rows_a, ssa)
    scat_wait(rows_b, ssb)
    plsc.subcore_barrier()

    @pl.when(sid == 0)
    def _():
      pltpu.sync_copy(acc_sh, out.at[:, cols])

  return pl.kernel(
      body,
      out_type=jax.ShapeDtypeStruct((np_total, 2 * wh), jnp.float32),
      mesh=mesh,
      scratch_types=scratch,
      compiler_params=pltpu.CompilerParams(use_tc_tiling_on_sc=False),
  )


def _dinv_block(degp_ref):
  deg = degp_ref[0, :, :1] + degp_ref[1, :, :1] + 1.0
  return lax.rsqrt(deg)


def _tca_body(x_ref, w_ref, out_ref):
  out_ref[...] = jnp.dot(x_ref[...], w_ref[...],
                         preferred_element_type=jnp.float32)


def _tc1_body(n_real, degp_ref, hm_ref, out_ref, dinv_ref):
  i = pl.program_id(0)
  rb = out_ref.shape[0]
  dinv = _dinv_block(degp_ref)
  rows = i * rb + lax.broadcasted_iota(jnp.int32, (rb, 1), 0)
  out_ref[...] = jnp.where(rows < n_real, hm_ref[...] * dinv, 0.0)
  dinv_ref[...] = jnp.broadcast_to(dinv, out_ref.shape)


def _tc2_body(n_real, dinv_ref, agg_ref, h1s_ref, w_ref, b_ref, out_ref):
  i = pl.program_id(0)
  rb = out_ref.shape[0]
  dinv = dinv_ref[...]
  a = agg_ref[...] + h1s_ref[...]
  hh = jnp.maximum(a * dinv + b_ref[...], 0.0)
  rows = i * rb + lax.broadcasted_iota(jnp.int32, (rb, 1), 0)
  hh = jnp.where(rows < n_real, hh, 0.0)
  out_ref[...] = jnp.dot(hh, w_ref[...], preferred_element_type=jnp.float32) * dinv


def _tc3_body(n_real, ng, dinv_ref, agg_ref, h2s_ref, b2_ref,
              fc1w_ref, fc1b_ref, fc2w_ref, fc2b_ref, out_ref, acc_ref):
  i = pl.program_id(0)
  dinv = dinv_ref[...]
  a = agg_ref[...] + h2s_ref[...]
  hh = jnp.maximum(a * dinv + b2_ref[...], 0.0)
  rb = h2s_ref.shape[0]
  rows = i * rb + lax.broadcasted_iota(jnp.int32, (rb, 1), 0)
  hh = jnp.where(rows < n_real, hh, 0.0)
  part = jnp.sum(hh, axis=0, keepdims=True)

  @pl.when(i == 0)
  def _():
    acc_ref[...] = part

  @pl.when(i > 0)
  def _():
    acc_ref[...] = acc_ref[...] + part

  @pl.when(i == ng - 1)
  def _():
    g = acc_ref[...] * (1.0 / n_real)
    g1 = jnp.maximum(
        jnp.dot(g, fc1w_ref[...], preferred_element_type=jnp.float32)
        + fc1b_ref[...], 0.0)
    logits = (jnp.dot(g1, fc2w_ref[...], preferred_element_type=jnp.float32)
              + fc2b_ref[...])
    out_ref[...] = 1.0 / (1.0 + jnp.exp(-logits))


def kernel(x, edge_index, W1, b1, W2, b2, fc1_w, fc1_b, fc2_w, fc2_b):
  n, d = x.shape
  h = W1.shape[1]
  o = fc2_w.shape[1]
  e = edge_index.shape[1]
  wh = h // 2

  np_total = ((n + 1 + _RB - 1) // _RB) * _RB
  rows_per_tile = np_total // _NS
  ng = np_total // _RB
  nt = _NC * _NS

  ec = (e + _EB - 1) // _EB
  ei = edge_index.astype(jnp.int32)
  if ec * _EB != e:
    ei = jnp.concatenate(
        [ei, jnp.full((2, ec * _EB - e), n, dtype=jnp.int32)], axis=1)
  ei3 = ei.reshape(2, ec, _EB)

  zrows_d = jnp.zeros((rows_per_tile, _DEGW), jnp.float32)
  ones_rows = jnp.ones((_EB, _DEGW), jnp.float32)
  zacc = jnp.zeros((np_total, wh), jnp.float32)

  deg_fn = _sc_deg(np_total, ec)
  agg_fn = _sc_agg(np_total, wh, ec)

  hm = pl.pallas_call(
      _tca_body,
      grid=(ng,),
      in_specs=[
          pl.BlockSpec((_RB, d), lambda i: (i, 0)),
          pl.BlockSpec((d, h), lambda i: (0, 0)),
      ],
      out_specs=pl.BlockSpec((_RB, h), lambda i: (i, 0)),
      out_shape=jax.ShapeDtypeStruct((np_total, h), jnp.float32),
  )(x, W1)

  degp = deg_fn(ei3, zrows_d, ones_rows)

  h1s, dinv_b = pl.pallas_call(
      functools.partial(_tc1_body, n),
      grid=(ng,),
      in_specs=[
          pl.BlockSpec((_NC, _RB, _DEGW), lambda i: (0, i, 0)),
          pl.BlockSpec((_RB, h), lambda i: (i, 0)),
      ],
      out_specs=[
          pl.BlockSpec((_RB, h), lambda i: (i, 0)),
          pl.BlockSpec((_RB, h), lambda i: (i, 0)),
      ],
      out_shape=[
          jax.ShapeDtypeStruct((np_total, h), jnp.float32),
          jax.ShapeDtypeStruct((np_total, h), jnp.float32),
      ],
  )(degp, hm)

  agg1 = agg_fn(h1s, ei3, zacc)

  h2s = pl.pallas_call(
      functools.partial(_tc2_body, n),
      grid=(ng,),
      in_specs=[
          pl.BlockSpec((_RB, h), lambda i: (i, 0)),
          pl.BlockSpec((_RB, h), lambda i: (i, 0)),
          pl.BlockSpec((_RB, h), lambda i: (i, 0)),
          pl.BlockSpec((h, h), lambda i: (0, 0)),
          pl.BlockSpec((1, h), lambda i: (0, 0)),
      ],
      out_specs=pl.BlockSpec((_RB, h), lambda i: (i, 0)),
      out_shape=jax.ShapeDtypeStruct((np_total, h), jnp.float32),
  )(dinv_b, agg1, h1s, W2, b1.reshape(1, h))

  agg2 = agg_fn(h2s, ei3, zacc)

  out = pl.pallas_call(
      functools.partial(_tc3_body, n, ng),
      grid=(ng,),
      in_specs=[
          pl.BlockSpec((_RB, h), lambda i: (i, 0)),
          pl.BlockSpec((_RB, h), lambda i: (i, 0)),
          pl.BlockSpec((_RB, h), lambda i: (i, 0)),
          pl.BlockSpec((1, h), lambda i: (0, 0)),
          pl.BlockSpec((h, h), lambda i: (0, 0)),
          pl.BlockSpec((1, h), lambda i: (0, 0)),
          pl.BlockSpec((h, o), lambda i: (0, 0)),
          pl.BlockSpec((1, o), lambda i: (0, 0)),
      ],
      out_specs=pl.BlockSpec((1, o), lambda i: (0, 0)),
      out_shape=jax.ShapeDtypeStruct((1, o), jnp.float32),
      scratch_shapes=[pltpu.VMEM((1, h), jnp.float32)],
  )(dinv_b, agg2, h2s, b2.reshape(1, h), fc1_w, fc1_b.reshape(1, h),
    fc2_w, fc2_b.reshape(1, o))

  return out.reshape(o)


# speedup vs baseline: 1.0068x; 1.0068x over previous
import functools

import jax
import jax.numpy as jnp
from jax import lax
from jax.experimental import pallas as pl
from jax.experimental.pallas import tpu as pltpu
from jax.experimental.pallas import tpu_sc as plsc

_NC = 2
_NS = 16
_EB = 128
_DEGW = 16
_RB = 5120


def _sc_deg(np_total, num_chunks):
  rows_per_tile = np_total // _NS
  mesh = plsc.VectorSubcoreMesh(core_axis_name="c", subcore_axis_name="s")
  ntg = _NC * _NS
  maxc_s = num_chunks - (num_chunks * (ntg - 1)) // ntg

  scratch = [
      pltpu.VMEM((maxc_s, _EB), jnp.int32),
      pltpu.VMEM((_EB, _DEGW), jnp.float32),
      pltpu.VMEM((rows_per_tile, _DEGW), jnp.float32),
      pltpu.VMEM_SHARED((np_total, _DEGW), jnp.float32),
      pltpu.SemaphoreType.DMA,
      pltpu.SemaphoreType.DMA,
  ]

  nt = _NC * _NS
  maxc = num_chunks - (num_chunks * (nt - 1)) // nt

  def body(ei3, zrows, orows, out, dst_all, rows_v, stage_v, acc_sh,
           ssa, ssb):
    cid = lax.axis_index("c")
    sid = lax.axis_index("s")
    tid = cid * _NS + sid
    row0 = sid * rows_per_tile
    tile_rows = pl.ds(row0, rows_per_tile)
    start = (num_chunks * tid) // nt
    cnt = (num_chunks * (tid + 1)) // nt - start

    pltpu.sync_copy(zrows, stage_v)
    pltpu.sync_copy(stage_v, acc_sh.at[tile_rows])
    pltpu.sync_copy(orows, rows_v)
    pltpu.sync_copy(ei3.at[1, pl.ds(start, maxc)], dst_all)
    plsc.subcore_barrier()

    def scat(c, sem):
      pltpu.async_copy(rows_v, acc_sh.at[dst_all.at[c]], sem, add=True)

    def scat_wait(sem):
      pltpu.make_async_copy(rows_v, acc_sh.at[dst_all.at[0]], sem).wait()

    def pair(j, carry):
      c0 = 2 * j

      @pl.when(j > 0)
      def _():
        scat_wait(ssa)
      scat(c0, ssa)

      @pl.when(j > 0)
      def _():
        scat_wait(ssb)
      scat(c0 + 1, ssb)
      return carry

    lax.fori_loop(0, cnt // 2, pair, 0)

    @pl.when(cnt % 2 == 1)
    def _():
      scat_wait(ssa)
      scat(cnt - 1, ssa)
    scat_wait(ssa)
    scat_wait(ssb)
    plsc.subcore_barrier()

    pltpu.sync_copy(acc_sh.at[tile_rows], stage_v)
    pltpu.sync_copy(stage_v, out.at[cid, tile_rows])

  return pl.kernel(
      body,
      out_type=jax.ShapeDtypeStruct((_NC, np_total, _DEGW), jnp.float32),
      mesh=mesh,
      scratch_types=scratch,
      compiler_params=pltpu.CompilerParams(use_tc_tiling_on_sc=False),
  )


def _sc_agg(np_total, wh, num_chunks):
  mesh = plsc.VectorSubcoreMesh(core_axis_name="c", subcore_axis_name="s")

  maxc = num_chunks - (num_chunks * (_NS - 1)) // _NS

  scratch = [
      pltpu.VMEM((maxc, _EB), jnp.int32),
      pltpu.VMEM((maxc, _EB), jnp.int32),
      pltpu.VMEM((_EB, wh), jnp.float32),
      pltpu.VMEM((_EB, wh), jnp.float32),
      pltpu.VMEM_SHARED((np_total, wh), jnp.float32),
      pltpu.VMEM_SHARED((np_total, wh), jnp.float32),
      pltpu.SemaphoreType.DMA,
      pltpu.SemaphoreType.DMA,
      pltpu.SemaphoreType.DMA,
      pltpu.SemaphoreType.DMA,
  ]

  def body(table, ei3, zacc, out,
           src_all, dst_all, rows_a, rows_b, table_sh, acc_sh,
           gsa, gsb, ssa, ssb):
    cid = lax.axis_index("c")
    sid = lax.axis_index("s")
    cols = pl.ds(cid * wh, wh)
    start = (num_chunks * sid) // _NS
    cnt = (num_chunks * (sid + 1)) // _NS - start

    @pl.when(sid == 0)
    def _():
      pltpu.sync_copy(zacc, acc_sh)
      pltpu.sync_copy(table.at[:, cols], table_sh)

    pltpu.sync_copy(ei3.at[0, pl.ds(start, maxc)], src_all)
    pltpu.sync_copy(ei3.at[1, pl.ds(start, maxc)], dst_all)
    plsc.subcore_barrier()

    def gath(c, rows, sem):
      pltpu.async_copy(table_sh.at[src_all.at[c]], rows, sem)

    def gath_wait(rows, sem):
      pltpu.make_async_copy(table_sh.at[src_all.at[0]], rows, sem).wait()

    def scat(c, rows, sem):
      pltpu.async_copy(rows, acc_sh.at[dst_all.at[c]], sem, add=True)

    def scat_wait(rows, sem):
      pltpu.make_async_copy(rows, acc_sh.at[dst_all.at[0]], sem).wait()

    def pair(j, carry):
      c0 = 2 * j
      c1 = c0 + 1

      @pl.when(j > 0)
      def _():
        scat_wait(rows_a, ssa)
      gath(c0, rows_a, gsa)

      @pl.when(j > 0)
      def _():
        scat_wait(rows_b, ssb)
      gath(c1, rows_b, gsb)

      gath_wait(rows_a, gsa)
      scat(c0, rows_a, ssa)
      gath_wait(rows_b, gsb)
      scat(c1, rows_b, ssb)
      return carry

    lax.fori_loop(0, cnt // 2, pair, 0)

    @pl.when(cnt % 2 == 1)
    def _():
      scat_wait(rows_a, ssa)
      gath(cnt - 1, rows_a, gsa)
      gath_wait(rows_a, gsa)
      scat(cnt - 1, rows_a, ssa)
    scat_wait(rows_a, ssa)
    scat_wait(rows_b, ssb)
    plsc.subcore_barrier()

    @pl.when(sid == 0)
    def _():
      pltpu.sync_copy(acc_sh, out.at[:, cols])

  return pl.kernel(
      body,
      out_type=jax.ShapeDtypeStruct((np_total, 2 * wh), jnp.float32),
      mesh=mesh,
      scratch_types=scratch,
      compiler_params=pltpu.CompilerParams(use_tc_tiling_on_sc=False),
  )


def _dinv_block(degp_ref):
  deg = degp_ref[0, :, :1] + degp_ref[1, :, :1] + 1.0
  return lax.rsqrt(deg)


def _tc1_body(n_real, degp_ref, x_ref, w_ref, out_ref, dinv_ref):
  i = pl.program_id(0)
  rb = out_ref.shape[0]
  dinv = _dinv_block(degp_ref)
  hm = jnp.dot(x_ref[...], w_ref[...], preferred_element_type=jnp.float32)
  rows = i * rb + lax.broadcasted_iota(jnp.int32, (rb, 1), 0)
  out_ref[...] = jnp.where(rows < n_real, hm * dinv, 0.0)
  dinv_ref[...] = jnp.broadcast_to(dinv, out_ref.shape)


def _tc2_body(n_real, dinv_ref, agg_ref, h1s_ref, w_ref, b_ref, out_ref):
  i = pl.program_id(0)
  rb = out_ref.shape[0]
  dinv = dinv_ref[...]
  a = agg_ref[...] + h1s_ref[...]
  hh = jnp.maximum(a * dinv + b_ref[...], 0.0)
  rows = i * rb + lax.broadcasted_iota(jnp.int32, (rb, 1), 0)
  hh = jnp.where(rows < n_real, hh, 0.0)
  out_ref[...] = jnp.dot(hh, w_ref[...], preferred_element_type=jnp.float32) * dinv


def _tc3_body(n_real, ng, dinv_ref, agg_ref, h2s_ref, b2_ref,
              fc1w_ref, fc1b_ref, fc2w_ref, fc2b_ref, out_ref, acc_ref):
  i = pl.program_id(0)
  dinv = dinv_ref[...]
  a = agg_ref[...] + h2s_ref[...]
  hh = jnp.maximum(a * dinv + b2_ref[...], 0.0)
  rb = h2s_ref.shape[0]
  rows = i * rb + lax.broadcasted_iota(jnp.int32, (rb, 1), 0)
  hh = jnp.where(rows < n_real, hh, 0.0)
  part = jnp.sum(hh, axis=0, keepdims=True)

  @pl.when(i == 0)
  def _():
    acc_ref[...] = part

  @pl.when(i > 0)
  def _():
    acc_ref[...] = acc_ref[...] + part

  @pl.when(i == ng - 1)
  def _():
    g = acc_ref[...] * (1.0 / n_real)
    g1 = jnp.maximum(
        jnp.dot(g, fc1w_ref[...], preferred_element_type=jnp.float32)
        + fc1b_ref[...], 0.0)
    logits = (jnp.dot(g1, fc2w_ref[...], preferred_element_type=jnp.float32)
              + fc2b_ref[...])
    out_ref[...] = 1.0 / (1.0 + jnp.exp(-logits))


def kernel(x, edge_index, W1, b1, W2, b2, fc1_w, fc1_b, fc2_w, fc2_b):
  n, d = x.shape
  h = W1.shape[1]
  o = fc2_w.shape[1]
  e = edge_index.shape[1]
  wh = h // 2

  np_total = ((n + 1 + _RB - 1) // _RB) * _RB
  rows_per_tile = np_total // _NS
  ng = np_total // _RB
  nt = _NC * _NS

  ec = (e + _EB - 1) // _EB
  ei = edge_index.astype(jnp.int32)
  if ec * _EB != e:
    ei = jnp.concatenate(
        [ei, jnp.full((2, ec * _EB - e), n, dtype=jnp.int32)], axis=1)
  ei3 = ei.reshape(2, ec, _EB)

  zrows_d = jnp.zeros((rows_per_tile, _DEGW), jnp.float32)
  ones_rows = jnp.ones((_EB, _DEGW), jnp.float32)
  zacc = jnp.zeros((np_total, wh), jnp.float32)

  deg_fn = _sc_deg(np_total, ec)
  agg_fn = _sc_agg(np_total, wh, ec)

  degp = deg_fn(ei3, zrows_d, ones_rows)

  h1s, dinv_b = pl.pallas_call(
      functools.partial(_tc1_body, n),
      grid=(ng,),
      in_specs=[
          pl.BlockSpec((_NC, _RB, _DEGW), lambda i: (0, i, 0)),
          pl.BlockSpec((_RB, d), lambda i: (i, 0)),
          pl.BlockSpec((d, h), lambda i: (0, 0)),
      ],
      out_specs=[
          pl.BlockSpec((_RB, h), lambda i: (i, 0)),
          pl.BlockSpec((_RB, h), lambda i: (i, 0)),
      ],
      out_shape=[
          jax.ShapeDtypeStruct((np_total, h), jnp.float32),
          jax.ShapeDtypeStruct((np_total, h), jnp.float32),
      ],
  )(degp, x, W1)

  agg1 = agg_fn(h1s, ei3, zacc)

  h2s = pl.pallas_call(
      functools.partial(_tc2_body, n),
      grid=(ng,),
      in_specs=[
          pl.BlockSpec((_RB, h), lambda i: (i, 0)),
          pl.BlockSpec((_RB, h), lambda i: (i, 0)),
          pl.BlockSpec((_RB, h), lambda i: (i, 0)),
          pl.BlockSpec((h, h), lambda i: (0, 0)),
          pl.BlockSpec((1, h), lambda i: (0, 0)),
      ],
      out_specs=pl.BlockSpec((_RB, h), lambda i: (i, 0)),
      out_shape=jax.ShapeDtypeStruct((np_total, h), jnp.float32),
  )(dinv_b, agg1, h1s, W2, b1.reshape(1, h))

  agg2 = agg_fn(h2s, ei3, zacc)

  out = pl.pallas_call(
      functools.partial(_tc3_body, n, ng),
      grid=(ng,),
      in_specs=[
          pl.BlockSpec((_RB, h), lambda i: (i, 0)),
          pl.BlockSpec((_RB, h), lambda i: (i, 0)),
          pl.BlockSpec((_RB, h), lambda i: (i, 0)),
          pl.BlockSpec((1, h), lambda i: (0, 0)),
          pl.BlockSpec((h, h), lambda i: (0, 0)),
          pl.BlockSpec((1, h), lambda i: (0, 0)),
          pl.BlockSpec((h, o), lambda i: (0, 0)),
          pl.BlockSpec((1, o), lambda i: (0, 0)),
      ],
      out_specs=pl.BlockSpec((1, o), lambda i: (0, 0)),
      out_shape=jax.ShapeDtypeStruct((1, o), jnp.float32),
      scratch_shapes=[pltpu.VMEM((1, h), jnp.float32)],
  )(dinv_b, agg2, h2s, b2.reshape(1, h), fc1_w, fc1_b.reshape(1, h),
    fc2_w, fc2_b.reshape(1, o))

  return out.reshape(o)
